# baseline (speedup 1.0000x reference)
import functools

import jax
import jax.numpy as jnp
from jax import lax
from jax.experimental import pallas as pl
from jax.experimental.pallas import tpu as pltpu
from jax.experimental.pallas import tpu_sc as plsc

DIM = 128
NC = 2
NS = 16
NW = NC * NS
CHUNK = 128

NBUF = 7
LOOKAHEAD = 6


def _gather_rope(idx, cos_cached, sin_cached, n_total):
    n_chunks = n_total // (NW * CHUNK)
    n_steps = 2 * n_chunks
    mesh = plsc.VectorSubcoreMesh(core_axis_name="c", subcore_axis_name="s")

    @functools.partial(
        pl.kernel,
        mesh=mesh,
        out_type=(
            jax.ShapeDtypeStruct((n_total, DIM), jnp.float32),
            jax.ShapeDtypeStruct((n_total, DIM), jnp.float32),
        ),
        scratch_types=[
            pltpu.VMEM((n_chunks * CHUNK,), jnp.int32),
            pltpu.VMEM((NBUF, CHUNK, DIM), jnp.float32),
            *([pltpu.SemaphoreType.DMA] * NBUF),
            *([pltpu.SemaphoreType.DMA] * NBUF),
        ],
    )
    def k(cos_hbm, sin_hbm, idx_hbm, cos_out, sin_out, idx_v, bufs, *sems):
        gsem, ssem = sems[:NBUF], sems[NBUF:]
        wid = lax.axis_index("s") * NC + lax.axis_index("c")
        base = wid * (n_chunks * CHUNK)
        pltpu.sync_copy(idx_hbm.at[pl.ds(base, n_chunks * CHUNK)], idx_v)

        def src(step):
            tab = cos_hbm if step < n_chunks else sin_hbm
            return tab.at[idx_v.at[pl.ds((step % n_chunks) * CHUNK, CHUNK)]]

        def dst(step):
            out = cos_out if step < n_chunks else sin_out
            return out.at[pl.ds(base + (step % n_chunks) * CHUNK, CHUNK)]

        stores = [None] * n_steps
        gathers = [None] * n_steps
        for t in range(LOOKAHEAD):
            gathers[t] = pltpu.async_copy(src(t), bufs.at[t % NBUF], gsem[t % NBUF])
        for s in range(n_steps):
            b = s % NBUF
            gathers[s].wait()
            stores[s] = pltpu.async_copy(bufs.at[b], dst(s), ssem[b])
            t = s + LOOKAHEAD
            if t < n_steps:
                bt = t % NBUF
                if t >= NBUF:
                    stores[t - NBUF].wait()
                gathers[t] = pltpu.async_copy(src(t), bufs.at[bt], gsem[bt])
        for s in range(n_steps - NBUF, n_steps):
            stores[s].wait()

    return k(cos_cached, sin_cached, idx)


def kernel(x, position_ids, cos_cached, sin_cached):
    b, s = position_ids.shape
    n_total = b * s
    idx = position_ids.astype(jnp.int32).reshape(n_total)
    cos_flat, sin_flat = _gather_rope(idx, cos_cached, sin_cached, n_total)
    cos = cos_flat.reshape(b, 1, s, DIM)
    sin = sin_flat.reshape(b, 1, s, DIM)
    return (cos, sin)


# speedup vs baseline: 1.0036x; 1.0036x over previous
import functools

import jax
import jax.numpy as jnp
from jax import lax
from jax.experimental import pallas as pl
from jax.experimental.pallas import tpu as pltpu
from jax.experimental.pallas import tpu_sc as plsc

DIM = 128
NC = 2
NS = 16
NW = NC * NS
CHUNK = 128

NBUF = 7
LOOKAHEAD = 6


def _gather_rope(idx, cos_cached, sin_cached, n_total):
    n_chunks = n_total // (NW * CHUNK)
    n_steps = 2 * n_chunks
    mesh = plsc.VectorSubcoreMesh(core_axis_name="c", subcore_axis_name="s")

    @functools.partial(
        pl.kernel,
        mesh=mesh,
        compiler_params=pltpu.CompilerParams(disable_semaphore_checks=True),
        out_type=(
            jax.ShapeDtypeStruct((n_total, DIM), jnp.float32),
            jax.ShapeDtypeStruct((n_total, DIM), jnp.float32),
        ),
        scratch_types=[
            pltpu.VMEM((n_chunks * CHUNK,), jnp.int32),
            pltpu.VMEM((NBUF, CHUNK, DIM), jnp.float32),
            *([pltpu.SemaphoreType.DMA] * NBUF),
            *([pltpu.SemaphoreType.DMA] * NBUF),
        ],
    )
    def k(cos_hbm, sin_hbm, idx_hbm, cos_out, sin_out, idx_v, bufs, *sems):
        gsem, ssem = sems[:NBUF], sems[NBUF:]
        wid = lax.axis_index("s") * NC + lax.axis_index("c")
        base = wid * (n_chunks * CHUNK)
        pltpu.sync_copy(idx_hbm.at[pl.ds(base, n_chunks * CHUNK)], idx_v)

        def src(step):
            tab = cos_hbm if step < n_chunks else sin_hbm
            return tab.at[idx_v.at[pl.ds((step % n_chunks) * CHUNK, CHUNK)]]

        def dst(step):
            out = cos_out if step < n_chunks else sin_out
            return out.at[pl.ds(base + (step % n_chunks) * CHUNK, CHUNK)]

        stores = [None] * n_steps
        gathers = [None] * n_steps
        for t in range(LOOKAHEAD):
            gathers[t] = pltpu.async_copy(src(t), bufs.at[t % NBUF], gsem[t % NBUF])
        for s in range(n_steps):
            b = s % NBUF
            gathers[s].wait()
            stores[s] = pltpu.async_copy(bufs.at[b], dst(s), ssem[b])
            t = s + LOOKAHEAD
            if t < n_steps:
                bt = t % NBUF
                if t >= NBUF:
                    stores[t - NBUF].wait()
                gathers[t] = pltpu.async_copy(src(t), bufs.at[bt], gsem[bt])
        for s in range(n_steps - NBUF, n_steps):
            stores[s].wait()

    return k(cos_cached, sin_cached, idx)


def kernel(x, position_ids, cos_cached, sin_cached):
    b, s = position_ids.shape
    n_total = b * s
    idx = position_ids.astype(jnp.int32).reshape(n_total)
    cos_flat, sin_flat = _gather_rope(idx, cos_cached, sin_cached, n_total)
    cos = cos_flat.reshape(b, 1, s, DIM)
    sin = sin_flat.reshape(b, 1, s, DIM)
    return (cos, sin)
